# baseline (speedup 1.0000x reference)
import jax
import jax.numpy as jnp
from jax.experimental import pallas as pl
from jax.experimental.pallas import tpu as pltpu

_V = 1000
_C = 64
_K = 20
_A = 3.0
_MIX = 0.05
_TB = 16
_TG = 4
_T = 192
_NG = _TG * _C


def _build_ant(e1_ref, e2_ref, w1_ref, b1_ref, w2_ref, b2_ref, ant_ref):
    dn_nt = (((1,), (1,)), ((), ()))
    f32 = jnp.float32

    def nv(e, w, b):
        return jnp.tanh(_A * (jax.lax.dot_general(
            e, w, dn_nt, preferred_element_type=f32) + b))

    nv1 = nv(e1_ref[...], w1_ref[...], b1_ref[...])
    nv2 = nv(e2_ref[...], w2_ref[...], b2_ref[...])
    g = (jax.lax.dot_general(nv2, nv1, dn_nt, preferred_element_type=f32)
         - jax.lax.dot_general(nv1, nv2, dn_nt, preferred_element_type=f32))
    adjt = jnp.maximum(jnp.tanh(_A * g), 0.0)
    bits = jax.lax.bitcast_convert_type(adjt, jnp.int32)

    def step(t, ans):
        cand = ans | jax.lax.shift_left(jnp.int32(1), 29 - t)
        cnt = jnp.sum((bits >= cand).astype(jnp.int32), axis=0, keepdims=True)
        return jnp.where(cnt >= _K, cand, ans)

    ans = jax.lax.fori_loop(0, 30, step, jnp.zeros((1, _V), jnp.int32))
    gt = bits > ans
    tie = bits == ans
    cnt_gt = jnp.sum(gt.astype(jnp.int32), axis=0, keepdims=True)
    rowi = jax.lax.broadcasted_iota(jnp.int32, (_V, _V), 0)
    coli = jax.lax.broadcasted_iota(jnp.int32, (_V, _V), 1)
    ltm = (coli < rowi).astype(f32)
    prefix = jax.lax.dot_general(ltm, tie.astype(f32), (((1,), (0,)), ((), ())),
                                 preferred_element_type=f32)
    keep = gt | (tie & (prefix < (_K - cnt_gt).astype(f32)))
    madj = jnp.where(keep, adjt, 0.0)
    madj = madj + (rowi == coli).astype(f32)
    d = jnp.sum(madj, axis=0, keepdims=True)
    ant_ref[...] = (madj / d).astype(jnp.bfloat16)


def _body(e1_ref, e2_ref, w1_ref, b1_ref, w2_ref, b2_ref, x_ref,
          wa_ref, wb_ref, wc_ref, bm_ref, y_ref, ant_ref, a2_ref):
    i = pl.program_id(0)
    dn = (((1,), (0,)), ((), ()))
    f32 = jnp.float32

    @pl.when(i == 0)
    def _():
        _build_ant(e1_ref, e2_ref, w1_ref, b1_ref, w2_ref, b2_ref, ant_ref)
        a = ant_ref[...]
        a2_ref[...] = jax.lax.dot_general(
            a, a, dn, preferred_element_type=f32).astype(jnp.bfloat16)

    ant = ant_ref[...]
    a2 = a2_ref[...]
    for g in range(_TB // _TG):
        xgb = jnp.concatenate(
            [x_ref[g * _TG + j] for j in range(_TG)], axis=1)
        p = jax.lax.dot_general(ant, xgb, dn, preferred_element_type=f32)
        r = jax.lax.dot_general(a2, xgb, dn, preferred_element_type=f32)
        y = (jax.lax.dot_general(xgb, wa_ref[...], dn,
                                 preferred_element_type=f32)
             + jax.lax.dot_general(p.astype(jnp.bfloat16), wb_ref[...], dn,
                                   preferred_element_type=f32)
             + jax.lax.dot_general(r.astype(jnp.bfloat16), wc_ref[...], dn,
                                   preferred_element_type=f32)
             + bm_ref[...])
        for j in range(_TG):
            y_ref[g * _TG + j] = y[:, j * _C:(j + 1) * _C]


def kernel(x, emb1, emb2, W1, b1, W2, b2, Wmlp, bmlp):
    f32 = jnp.float32
    bf16 = jnp.bfloat16
    b1r = b1.reshape(1, _C).astype(f32)
    b2r = b2.reshape(1, _C).astype(f32)
    eye4 = jnp.eye(_TG, dtype=f32)
    w0t, w1t, w2t = (Wmlp[:, :_C].T, Wmlp[:, _C:2 * _C].T, Wmlp[:, 2 * _C:].T)
    a_, b_ = _MIX, 1.0 - _MIX
    g0 = w0t + a_ * (w1t + w2t)
    g1 = b_ * w1t + a_ * b_ * w2t
    g2 = (b_ * b_) * w2t
    w0bd = jnp.kron(eye4, g0).astype(bf16)
    w1bd = jnp.kron(eye4, g1).astype(bf16)
    w2bd = jnp.kron(eye4, g2).astype(bf16)
    bmbd = jnp.tile(bmlp.reshape(1, _C), (1, _TG)).astype(f32)

    xb = x.astype(bf16)
    cfull = lambda i: (0, 0)
    y = pl.pallas_call(
        _body,
        grid=(_T // _TB,),
        in_specs=[
            pl.BlockSpec((_V, _C), cfull),
            pl.BlockSpec((_V, _C), cfull),
            pl.BlockSpec((_C, _C), cfull),
            pl.BlockSpec((1, _C), cfull),
            pl.BlockSpec((_C, _C), cfull),
            pl.BlockSpec((1, _C), cfull),
            pl.BlockSpec((_TB, _V, _C), lambda i: (i, 0, 0)),
            pl.BlockSpec((_NG, _NG), cfull),
            pl.BlockSpec((_NG, _NG), cfull),
            pl.BlockSpec((_NG, _NG), cfull),
            pl.BlockSpec((1, _NG), cfull),
        ],
        out_specs=pl.BlockSpec((_TB, _V, _C), lambda i: (i, 0, 0)),
        out_shape=jax.ShapeDtypeStruct((_T, _V, _C), f32),
        scratch_shapes=[pltpu.VMEM((_V, _V), bf16),
                        pltpu.VMEM((_V, _V), bf16)],
        compiler_params=pltpu.CompilerParams(
            dimension_semantics=("arbitrary",)),
    )(emb1, emb2, W1, b1r, W2, b2r, xb, w0bd, w1bd, w2bd, bmbd)
    return y


# speedup vs baseline: 1.1909x; 1.1909x over previous
import jax
import jax.numpy as jnp
from jax.experimental import pallas as pl
from jax.experimental.pallas import tpu as pltpu

_V = 1000
_C = 64
_K = 20
_A = 3.0
_MIX = 0.05
_TB = 16
_TG = 4
_T = 192
_NG = _TG * _C


def _build_ant(e1_ref, e2_ref, w1_ref, b1_ref, w2_ref, b2_ref, ant_ref):
    dn_nt = (((1,), (1,)), ((), ()))
    f32 = jnp.float32

    def nv(e, w, b):
        return jnp.tanh(_A * (jax.lax.dot_general(
            e, w, dn_nt, preferred_element_type=f32) + b))

    nv1 = nv(e1_ref[...], w1_ref[...], b1_ref[...])
    nv2 = nv(e2_ref[...], w2_ref[...], b2_ref[...])
    g = (jax.lax.dot_general(nv2, nv1, dn_nt, preferred_element_type=f32)
         - jax.lax.dot_general(nv1, nv2, dn_nt, preferred_element_type=f32))
    adjt = jnp.maximum(jnp.tanh(_A * g), 0.0)
    bits = jax.lax.bitcast_convert_type(adjt, jnp.int32)

    def step(t, ans):
        cand = ans | jax.lax.shift_left(jnp.int32(1), 29 - t)
        cnt = jnp.sum((bits >= cand).astype(jnp.int32), axis=0, keepdims=True)
        return jnp.where(cnt >= _K, cand, ans)

    ans = jax.lax.fori_loop(0, 30, step, jnp.zeros((1, _V), jnp.int32))
    gt = bits > ans
    tie = bits == ans
    cnt_gt = jnp.sum(gt.astype(jnp.int32), axis=0, keepdims=True)
    rowi = jax.lax.broadcasted_iota(jnp.int32, (_V, _V), 0)
    coli = jax.lax.broadcasted_iota(jnp.int32, (_V, _V), 1)
    ltm = (coli < rowi).astype(f32)
    prefix = jax.lax.dot_general(ltm, tie.astype(f32), (((1,), (0,)), ((), ())),
                                 preferred_element_type=f32)
    keep = gt | (tie & (prefix < (_K - cnt_gt).astype(f32)))
    madj = jnp.where(keep, adjt, 0.0)
    madj = madj + (rowi == coli).astype(f32)
    d = jnp.sum(madj, axis=0, keepdims=True)
    ant_ref[...] = (madj / d).astype(jnp.bfloat16)


def _body(e1_ref, e2_ref, w1_ref, b1_ref, w2_ref, b2_ref, x_ref,
          wa_ref, wb_ref, wc_ref, bm_ref, y_ref, ant_ref, a2_ref):
    i = pl.program_id(0)
    dn = (((1,), (0,)), ((), ()))
    f32 = jnp.float32

    @pl.when(i == 0)
    def _():
        _build_ant(e1_ref, e2_ref, w1_ref, b1_ref, w2_ref, b2_ref, ant_ref)
        a = ant_ref[...]
        a2_ref[...] = jax.lax.dot_general(
            a, a, dn, preferred_element_type=f32).astype(jnp.bfloat16)

    ant = ant_ref[...]
    a2 = a2_ref[...]
    for g in range(_TB // _TG):
        xgb = jnp.concatenate(
            [x_ref[pl.ds((g * _TG + j) * _V, _V), :] for j in range(_TG)],
            axis=1)
        p = jax.lax.dot_general(ant, xgb, dn, preferred_element_type=f32)
        r = jax.lax.dot_general(a2, xgb, dn, preferred_element_type=f32)
        y = (jax.lax.dot_general(xgb, wa_ref[...], dn,
                                 preferred_element_type=f32)
             + jax.lax.dot_general(p.astype(jnp.bfloat16), wb_ref[...], dn,
                                   preferred_element_type=f32)
             + jax.lax.dot_general(r.astype(jnp.bfloat16), wc_ref[...], dn,
                                   preferred_element_type=f32)
             + bm_ref[...])
        for j in range(_TG):
            y_ref[pl.ds((g * _TG + j) * _V, _V), :] = y[:, j * _C:(j + 1) * _C]


def kernel(x, emb1, emb2, W1, b1, W2, b2, Wmlp, bmlp):
    f32 = jnp.float32
    bf16 = jnp.bfloat16
    b1r = b1.reshape(1, _C).astype(f32)
    b2r = b2.reshape(1, _C).astype(f32)
    eye4 = jnp.eye(_TG, dtype=f32)
    w0t, w1t, w2t = (Wmlp[:, :_C].T, Wmlp[:, _C:2 * _C].T, Wmlp[:, 2 * _C:].T)
    a_, b_ = _MIX, 1.0 - _MIX
    g0 = w0t + a_ * (w1t + w2t)
    g1 = b_ * w1t + a_ * b_ * w2t
    g2 = (b_ * b_) * w2t
    w0bd = jnp.kron(eye4, g0).astype(bf16)
    w1bd = jnp.kron(eye4, g1).astype(bf16)
    w2bd = jnp.kron(eye4, g2).astype(bf16)
    bmbd = jnp.tile(bmlp.reshape(1, _C), (1, _TG)).astype(f32)

    xb = x.astype(bf16).reshape(_T * _V, _C)
    cfull = lambda i: (0, 0)
    y = pl.pallas_call(
        _body,
        grid=(_T // _TB,),
        in_specs=[
            pl.BlockSpec((_V, _C), cfull),
            pl.BlockSpec((_V, _C), cfull),
            pl.BlockSpec((_C, _C), cfull),
            pl.BlockSpec((1, _C), cfull),
            pl.BlockSpec((_C, _C), cfull),
            pl.BlockSpec((1, _C), cfull),
            pl.BlockSpec((_TB * _V, _C), lambda i: (i, 0)),
            pl.BlockSpec((_NG, _NG), cfull),
            pl.BlockSpec((_NG, _NG), cfull),
            pl.BlockSpec((_NG, _NG), cfull),
            pl.BlockSpec((1, _NG), cfull),
        ],
        out_specs=pl.BlockSpec((_TB * _V, _C), lambda i: (i, 0)),
        out_shape=jax.ShapeDtypeStruct((_T * _V, _C), f32),
        scratch_shapes=[pltpu.VMEM((_V, _V), bf16),
                        pltpu.VMEM((_V, _V), bf16)],
        compiler_params=pltpu.CompilerParams(
            dimension_semantics=("arbitrary",)),
    )(emb1, emb2, W1, b1r, W2, b2r, xb, w0bd, w1bd, w2bd, bmbd)
    return y.reshape(_T, _V, _C)


# speedup vs baseline: 1.2401x; 1.0413x over previous
import jax
import jax.numpy as jnp
from jax.experimental import pallas as pl
from jax.experimental.pallas import tpu as pltpu

_V = 1000
_C = 64
_K = 20
_A = 3.0
_MIX = 0.05
_TB = 16
_TG = 4
_T = 192
_NG = _TG * _C


def _build_ant(e1_ref, e2_ref, w1_ref, b1_ref, w2_ref, b2_ref, ant_ref):
    dn_nt = (((1,), (1,)), ((), ()))
    f32 = jnp.float32

    def nv(e, w, b):
        return jnp.tanh(_A * (jax.lax.dot_general(
            e, w, dn_nt, preferred_element_type=f32) + b))

    nv1 = nv(e1_ref[...], w1_ref[...], b1_ref[...])
    nv2 = nv(e2_ref[...], w2_ref[...], b2_ref[...])
    g = (jax.lax.dot_general(nv2, nv1, dn_nt, preferred_element_type=f32)
         - jax.lax.dot_general(nv1, nv2, dn_nt, preferred_element_type=f32))
    adjt = jnp.maximum(jnp.tanh(_A * g), 0.0)
    bits = jax.lax.bitcast_convert_type(adjt, jnp.int32)

    def step(t, ans):
        cand = ans | jax.lax.shift_left(jnp.int32(1), 29 - t)
        cnt = jnp.sum((bits >= cand).astype(jnp.int32), axis=0, keepdims=True)
        return jnp.where(cnt >= _K, cand, ans)

    ans = jax.lax.fori_loop(0, 30, step, jnp.zeros((1, _V), jnp.int32))
    gt = bits > ans
    tie = bits == ans
    cnt_gt = jnp.sum(gt.astype(jnp.int32), axis=0, keepdims=True)
    rowi = jax.lax.broadcasted_iota(jnp.int32, (_V, _V), 0)
    coli = jax.lax.broadcasted_iota(jnp.int32, (_V, _V), 1)
    ltm = (coli < rowi).astype(f32)
    prefix = jax.lax.dot_general(ltm, tie.astype(f32), (((1,), (0,)), ((), ())),
                                 preferred_element_type=f32)
    keep = gt | (tie & (prefix < (_K - cnt_gt).astype(f32)))
    madj = jnp.where(keep, adjt, 0.0)
    madj = madj + (rowi == coli).astype(f32)
    d = jnp.sum(madj, axis=0, keepdims=True)
    ant_ref[...] = (madj / d).astype(jnp.bfloat16)


def _body(e1_ref, e2_ref, w1_ref, b1_ref, w2_ref, b2_ref, x_ref,
          wa_ref, wb_ref, wc_ref, bm_ref, y_ref, ant_ref, a2_ref):
    i = pl.program_id(0)
    dn = (((1,), (0,)), ((), ()))
    f32 = jnp.float32

    @pl.when(i == 0)
    def _():
        _build_ant(e1_ref, e2_ref, w1_ref, b1_ref, w2_ref, b2_ref, ant_ref)
        a = ant_ref[...]
        a2_ref[...] = jax.lax.dot_general(
            a, a, dn, preferred_element_type=f32).astype(jnp.bfloat16)

    ant = ant_ref[...]
    a2 = a2_ref[...]
    for g in range(_TB // _TG):
        xgb = jnp.concatenate(
            [x_ref[pl.ds((g * _TG + j) * _V, _V), :] for j in range(_TG)],
            axis=1).astype(jnp.bfloat16)
        p = jax.lax.dot_general(ant, xgb, dn, preferred_element_type=f32)
        r = jax.lax.dot_general(a2, xgb, dn, preferred_element_type=f32)
        y = (jax.lax.dot_general(xgb, wa_ref[...], dn,
                                 preferred_element_type=f32)
             + jax.lax.dot_general(p.astype(jnp.bfloat16), wb_ref[...], dn,
                                   preferred_element_type=f32)
             + jax.lax.dot_general(r.astype(jnp.bfloat16), wc_ref[...], dn,
                                   preferred_element_type=f32)
             + bm_ref[...])
        for j in range(_TG):
            y_ref[pl.ds((g * _TG + j) * _V, _V), :] = y[:, j * _C:(j + 1) * _C]


def kernel(x, emb1, emb2, W1, b1, W2, b2, Wmlp, bmlp):
    f32 = jnp.float32
    bf16 = jnp.bfloat16
    b1r = b1.reshape(1, _C).astype(f32)
    b2r = b2.reshape(1, _C).astype(f32)
    eye4 = jnp.eye(_TG, dtype=f32)
    w0t, w1t, w2t = (Wmlp[:, :_C].T, Wmlp[:, _C:2 * _C].T, Wmlp[:, 2 * _C:].T)
    a_, b_ = _MIX, 1.0 - _MIX
    g0 = w0t + a_ * (w1t + w2t)
    g1 = b_ * w1t + a_ * b_ * w2t
    g2 = (b_ * b_) * w2t
    w0bd = jnp.kron(eye4, g0).astype(bf16)
    w1bd = jnp.kron(eye4, g1).astype(bf16)
    w2bd = jnp.kron(eye4, g2).astype(bf16)
    bmbd = jnp.tile(bmlp.reshape(1, _C), (1, _TG)).astype(f32)

    xb = x.reshape(_T * _V, _C)
    cfull = lambda i: (0, 0)
    y = pl.pallas_call(
        _body,
        grid=(_T // _TB,),
        in_specs=[
            pl.BlockSpec((_V, _C), cfull),
            pl.BlockSpec((_V, _C), cfull),
            pl.BlockSpec((_C, _C), cfull),
            pl.BlockSpec((1, _C), cfull),
            pl.BlockSpec((_C, _C), cfull),
            pl.BlockSpec((1, _C), cfull),
            pl.BlockSpec((_TB * _V, _C), lambda i: (i, 0)),
            pl.BlockSpec((_NG, _NG), cfull),
            pl.BlockSpec((_NG, _NG), cfull),
            pl.BlockSpec((_NG, _NG), cfull),
            pl.BlockSpec((1, _NG), cfull),
        ],
        out_specs=pl.BlockSpec((_TB * _V, _C), lambda i: (i, 0)),
        out_shape=jax.ShapeDtypeStruct((_T * _V, _C), f32),
        scratch_shapes=[pltpu.VMEM((_V, _V), bf16),
                        pltpu.VMEM((_V, _V), bf16)],
        compiler_params=pltpu.CompilerParams(
            dimension_semantics=("arbitrary",)),
    )(emb1, emb2, W1, b1r, W2, b2r, xb, w0bd, w1bd, w2bd, bmbd)
    return y.reshape(_T, _V, _C)
